# baseline (speedup 1.0000x reference)
import functools

import jax
import jax.numpy as jnp
from jax import lax
from jax.experimental import pallas as pl
from jax.experimental.pallas import tpu as pltpu
from jax.experimental.pallas import tpu_sc as plsc

_TOP_K = 5
_BATCH = 1024
_VOCAB = 100000

_NC = 2
_NS = 16
_L = 16
_NW = _NC * _NS
_BPW = _BATCH // _NW

_BR = 32
_NRB = _BATCH // _BR


def _gather_body(pred_flat_hbm, target_hbm, t_hbm, tgt_v, idx_v, out_v, sem):
    wid = lax.axis_index("s") * _NC + lax.axis_index("c")
    base = wid * _BPW
    pltpu.sync_copy(target_hbm.at[pl.ds(base, _BPW)], tgt_v)
    for k in range(_BPW // _L):
        row = base + k * _L + lax.iota(jnp.int32, _L)
        idx_v[pl.ds(k * _L, _L)] = tgt_v[pl.ds(k * _L, _L)] + row * _VOCAB
    pltpu.async_copy(pred_flat_hbm.at[idx_v], out_v, sem).wait()
    pltpu.sync_copy(out_v, t_hbm.at[pl.ds(base, _BPW)])


@functools.cache
def _make_gather():
    return pl.kernel(
        _gather_body,
        out_type=jax.ShapeDtypeStruct((_BATCH,), jnp.float32),
        mesh=plsc.VectorSubcoreMesh(
            core_axis_name="c", subcore_axis_name="s",
            num_cores=_NC, num_subcores=_NS,
        ),
        scratch_types=[
            pltpu.VMEM((_BPW,), jnp.int32),
            pltpu.VMEM((_BPW,), jnp.int32),
            pltpu.VMEM((_BPW,), jnp.float32),
            pltpu.SemaphoreType.DMA,
        ],
    )


_NSPLIT = 4
_BRS = _BR // _NSPLIT


def _count_body(t_ref, tgt_ref, *rest):
    out_ref = rest[_NSPLIT]
    rb = pl.program_id(0)
    t = t_ref[...]
    tgt = tgt_ref[...]
    cols = lax.broadcasted_iota(jnp.int32, (_BRS, _VOCAB), 1)
    correct = jnp.float32(0.0)
    for h in range(_NSPLIT):
        x = rest[h][...]
        th = t[h * _BRS:(h + 1) * _BRS, :]
        tg = tgt[h * _BRS:(h + 1) * _BRS, :]
        thr = jnp.where(x > th, _VOCAB, jnp.where(x == th, tg, 0))
        cnt = jnp.sum((cols < thr).astype(jnp.int32), axis=1, keepdims=True)
        correct += jnp.sum((cnt < _TOP_K).astype(jnp.float32))

    @pl.when(rb == 0)
    def _init():
        out_ref[0, 0] = 0.0

    out_ref[0, 0] += 100.0 * correct / jnp.float32(_BATCH)


def _split_spec(h):
    return pl.BlockSpec((_BRS, _VOCAB), lambda rb: (rb * _NSPLIT + h, 0))


_count = pl.pallas_call(
    _count_body,
    grid=(_NRB,),
    in_specs=[
        pl.BlockSpec((_BR, 1), lambda rb: (rb, 0)),
        pl.BlockSpec((_BR, 1), lambda rb: (rb, 0)),
    ] + [_split_spec(h) for h in range(_NSPLIT)],
    out_specs=pl.BlockSpec(memory_space=pltpu.SMEM),
    out_shape=jax.ShapeDtypeStruct((1, 1), jnp.float32),
    compiler_params=pltpu.CompilerParams(
        dimension_semantics=(pltpu.ARBITRARY,),
        vmem_limit_bytes=100 * 1024 * 1024,
    ),
)


@jax.jit
def kernel(pred, target):
    target = target.astype(jnp.int32)
    t = _make_gather()(pred.reshape(-1), target)
    acc = _count(t.reshape(_BATCH, 1), target.reshape(_BATCH, 1),
                 *([pred] * _NSPLIT))
    return acc[0, 0]


# speedup vs baseline: 6.3233x; 6.3233x over previous
import functools

import jax
import jax.numpy as jnp
from jax import lax
from jax.experimental import pallas as pl
from jax.experimental.pallas import tpu as pltpu
from jax.experimental.pallas import tpu_sc as plsc

_TOP_K = 5
_BATCH = 1024
_VOCAB = 100000

_NC = 2
_NS = 16
_NW = _NC * _NS
_BPW = _BATCH // _NW

_BV = 4096
_NVB = (_VOCAB + _BV - 1) // _BV


def _gather_body(predT_hbm, target_hbm, g_hbm, idx_v, rows_v, sem):
    wid = lax.axis_index("s") * _NC + lax.axis_index("c")
    base = wid * _BPW
    pltpu.sync_copy(target_hbm.at[pl.ds(base, _BPW)], idx_v)
    pltpu.async_copy(predT_hbm.at[idx_v], rows_v, sem).wait()
    pltpu.sync_copy(rows_v, g_hbm.at[pl.ds(base, _BPW)])


@functools.cache
def _make_gather():
    return pl.kernel(
        _gather_body,
        out_type=jax.ShapeDtypeStruct((_BATCH, _BATCH), jnp.float32),
        mesh=plsc.VectorSubcoreMesh(
            core_axis_name="c", subcore_axis_name="s",
            num_cores=_NC, num_subcores=_NS,
        ),
        scratch_types=[
            pltpu.VMEM((_BPW,), jnp.int32),
            pltpu.VMEM((_BPW, _BATCH), jnp.float32),
            pltpu.SemaphoreType.DMA,
        ],
    )


def _diag_body(g_ref, t_ref):
    rows = lax.broadcasted_iota(jnp.int32, (_BATCH, _BATCH), 0)
    cols = lax.broadcasted_iota(jnp.int32, (_BATCH, _BATCH), 1)
    t_ref[...] = jnp.sum(
        jnp.where(rows == cols, g_ref[...], 0.0), axis=0, keepdims=True)


_diag = pl.pallas_call(
    _diag_body,
    out_shape=jax.ShapeDtypeStruct((1, _BATCH), jnp.float32),
)


def _count_body(t_ref, tgt_ref, predT_ref, out_ref, acc_ref):
    vb = pl.program_id(0)
    x = predT_ref[...]
    t = t_ref[...]
    tgt = tgt_ref[...]
    rows = vb * _BV + lax.broadcasted_iota(jnp.int32, (_BV, _BATCH), 0)
    thr = jnp.where(x > t, _VOCAB, jnp.where(x == t, tgt, 0))
    cnt = jnp.sum((rows < thr).astype(jnp.int32), axis=0, keepdims=True)

    @pl.when(vb == 0)
    def _init():
        acc_ref[...] = cnt

    @pl.when(vb != 0)
    def _accum():
        acc_ref[...] = acc_ref[...] + cnt

    @pl.when(vb == _NVB - 1)
    def _finish():
        correct = (acc_ref[...] < _TOP_K).astype(jnp.float32)
        out_ref[0, 0] = 100.0 * jnp.sum(correct) / jnp.float32(_BATCH)


_count = pl.pallas_call(
    _count_body,
    grid=(_NVB,),
    in_specs=[
        pl.BlockSpec((1, _BATCH), lambda vb: (0, 0)),
        pl.BlockSpec((1, _BATCH), lambda vb: (0, 0)),
        pl.BlockSpec((_BV, _BATCH), lambda vb: (vb, 0)),
    ],
    out_specs=pl.BlockSpec(memory_space=pltpu.SMEM),
    out_shape=jax.ShapeDtypeStruct((1, 1), jnp.float32),
    scratch_shapes=[pltpu.VMEM((1, _BATCH), jnp.int32)],
    compiler_params=pltpu.CompilerParams(
        dimension_semantics=(pltpu.ARBITRARY,),
        vmem_limit_bytes=100 * 1024 * 1024,
    ),
)


@jax.jit
def kernel(pred, target):
    target = target.astype(jnp.int32)
    predT = jnp.swapaxes(pred, 0, 1)
    g = _make_gather()(predT, target)
    t_row = _diag(g)
    acc = _count(t_row, target.reshape(1, _BATCH), predT)
    return acc[0, 0]


# speedup vs baseline: 6.3696x; 1.0073x over previous
import functools

import jax
import jax.numpy as jnp
from jax import lax
from jax.experimental import pallas as pl
from jax.experimental.pallas import tpu as pltpu
from jax.experimental.pallas import tpu_sc as plsc

_TOP_K = 5
_BATCH = 1024
_VOCAB = 100000

_NC = 2
_NS = 16
_NW = _NC * _NS
_BPW = _BATCH // _NW

_BV = 4096
_NVB = (_VOCAB + _BV - 1) // _BV


def _gather_body(predT_hbm, target_hbm, g_hbm, idx_v, rows_v, sem):
    wid = lax.axis_index("s") * _NC + lax.axis_index("c")
    base = wid * _BPW
    pltpu.sync_copy(target_hbm.at[pl.ds(base, _BPW)], idx_v)
    pltpu.async_copy(predT_hbm.at[idx_v], rows_v, sem).wait()
    pltpu.sync_copy(rows_v, g_hbm.at[pl.ds(base, _BPW)])


@functools.cache
def _make_gather():
    return pl.kernel(
        _gather_body,
        out_type=jax.ShapeDtypeStruct((_BATCH, _BATCH), jnp.float32),
        mesh=plsc.VectorSubcoreMesh(
            core_axis_name="c", subcore_axis_name="s",
            num_cores=_NC, num_subcores=_NS,
        ),
        scratch_types=[
            pltpu.VMEM((_BPW,), jnp.int32),
            pltpu.VMEM((_BPW, _BATCH), jnp.float32),
            pltpu.SemaphoreType.DMA,
        ],
    )


def _count_body(g_ref, tgt_ref, predT_ref, out_ref, t_ref, acc_ref):
    vb = pl.program_id(0)

    @pl.when(vb == 0)
    def _extract_t():
        rows = lax.broadcasted_iota(jnp.int32, (_BATCH, _BATCH), 0)
        cols = lax.broadcasted_iota(jnp.int32, (_BATCH, _BATCH), 1)
        t_ref[...] = jnp.sum(
            jnp.where(rows == cols, g_ref[...], 0.0), axis=0, keepdims=True)

    x = predT_ref[...]
    t = t_ref[...]
    tgt = tgt_ref[...]
    rows = vb * _BV + lax.broadcasted_iota(jnp.int32, (_BV, _BATCH), 0)
    thr = jnp.where(x > t, _VOCAB, jnp.where(x == t, tgt, 0))
    cnt = jnp.sum((rows < thr).astype(jnp.int32), axis=0, keepdims=True)

    @pl.when(vb == 0)
    def _init():
        acc_ref[...] = cnt

    @pl.when(vb != 0)
    def _accum():
        acc_ref[...] = acc_ref[...] + cnt

    @pl.when(vb == _NVB - 1)
    def _finish():
        correct = (acc_ref[...] < _TOP_K).astype(jnp.float32)
        out_ref[0, 0] = 100.0 * jnp.sum(correct) / jnp.float32(_BATCH)


_count = pl.pallas_call(
    _count_body,
    grid=(_NVB,),
    in_specs=[
        pl.BlockSpec((_BATCH, _BATCH), lambda vb: (0, 0)),
        pl.BlockSpec((1, _BATCH), lambda vb: (0, 0)),
        pl.BlockSpec((_BV, _BATCH), lambda vb: (vb, 0)),
    ],
    out_specs=pl.BlockSpec(memory_space=pltpu.SMEM),
    out_shape=jax.ShapeDtypeStruct((1, 1), jnp.float32),
    scratch_shapes=[pltpu.VMEM((1, _BATCH), jnp.float32),
                    pltpu.VMEM((1, _BATCH), jnp.int32)],
    compiler_params=pltpu.CompilerParams(
        dimension_semantics=(pltpu.ARBITRARY,),
        vmem_limit_bytes=100 * 1024 * 1024,
    ),
)


@jax.jit
def kernel(pred, target):
    target = target.astype(jnp.int32)
    predT = jnp.swapaxes(pred, 0, 1)
    g = _make_gather()(predT, target)
    acc = _count(g, target.reshape(1, _BATCH), predT)
    return acc[0, 0]
